# baseline (speedup 1.0000x reference)
import functools

import jax
import jax.numpy as jnp
from jax import lax
from jax.experimental import pallas as pl
from jax.experimental.pallas import tpu as pltpu
from jax.experimental.pallas import tpu_sc as plsc

NSUB = 4096
DIM = 256
K = 16
ALPHA = 3.0
BR = 256
_DOT_DIMS = (((1,), (1,)), ((), ()))


@functools.cache
def _make_sc_gather():
    info = plsc.get_sparse_core_info()
    nc, ns = info.num_cores, info.num_subcores
    nw = nc * ns
    bpw = NSUB // nw

    @functools.partial(
        pl.kernel,
        mesh=plsc.VectorSubcoreMesh(core_axis_name="c", subcore_axis_name="s"),
        out_type=[
            jax.ShapeDtypeStruct((NSUB, DIM), jnp.float32),
            jax.ShapeDtypeStruct((NSUB, DIM), jnp.float32),
        ],
        scratch_types=[
            pltpu.VMEM((bpw,), jnp.int32),
            pltpu.VMEM((bpw, DIM), jnp.float32),
            pltpu.VMEM((bpw, DIM), jnp.float32),
            pltpu.SemaphoreType.DMA,
            pltpu.SemaphoreType.DMA,
        ],
    )
    def sc_gather(emb1_hbm, emb2_hbm, idx_hbm, g1_hbm, g2_hbm,
                  idx_v, rows1_v, rows2_v, sem1, sem2):
        wid = lax.axis_index("s") * nc + lax.axis_index("c")
        base = wid * bpw
        pltpu.sync_copy(idx_hbm.at[pl.ds(base, bpw)], idx_v)
        c1 = pltpu.async_copy(emb1_hbm.at[idx_v], rows1_v, sem1)
        c2 = pltpu.async_copy(emb2_hbm.at[idx_v], rows2_v, sem2)
        c1.wait()
        c2.wait()
        pltpu.sync_copy(rows1_v, g1_hbm.at[pl.ds(base, bpw)])
        pltpu.sync_copy(rows2_v, g2_hbm.at[pl.ds(base, bpw)])

    return sc_gather


def _linear_body(g1_ref, g2_ref, w1_ref, b1_ref, w2_ref, b2_ref,
                 v1_ref, v2_ref):
    v1_ref[...] = jnp.tanh(ALPHA * (
        lax.dot_general(g1_ref[...], w1_ref[...], _DOT_DIMS,
                        preferred_element_type=jnp.float32)
        + b1_ref[...]))
    v2_ref[...] = jnp.tanh(ALPHA * (
        lax.dot_general(g2_ref[...], w2_ref[...], _DOT_DIMS,
                        preferred_element_type=jnp.float32)
        + b2_ref[...]))


def _linear(g1, g2, W1, b1, W2, b2):
    return pl.pallas_call(
        _linear_body,
        out_shape=[
            jax.ShapeDtypeStruct((NSUB, DIM), jnp.float32),
            jax.ShapeDtypeStruct((NSUB, DIM), jnp.float32),
        ],
    )(g1, g2, W1, b1.reshape(1, DIM), W2, b2.reshape(1, DIM))


def _main_body(v1_ref, v2_ref, v1l_ref, v2l_ref, noise_ref, out_ref):
    a = (lax.dot_general(v1l_ref[...], v2_ref[...], _DOT_DIMS,
                         preferred_element_type=jnp.float32)
         - lax.dot_general(v2l_ref[...], v1_ref[...], _DOT_DIMS,
                           preferred_element_type=jnp.float32))
    adj = jnp.maximum(jnp.tanh(ALPHA * a), 0.0)
    y = adj + noise_ref[...]
    x = y
    t = jnp.zeros((BR, 1), jnp.float32)
    cum = jnp.zeros((BR, 1), jnp.float32)
    for _ in range(K):
        m = jnp.max(x, axis=1, keepdims=True)
        eqm = x == m
        cnt = jnp.sum(eqm.astype(jnp.float32), axis=1, keepdims=True)
        t = jnp.where(cum < K, m, t)
        cum = cum + cnt
        x = jnp.where(eqm, -1.0, x)

    gt = y > t
    quota = K - jnp.sum(gt.astype(jnp.float32), axis=1, keepdims=True)
    eq = y == t
    pfx = _lane_cumsum(eq.astype(jnp.float32))
    sel = gt | (eq & (pfx <= quota))
    out_ref[...] = jnp.where(sel, adj, 0.0)


def _lane_cumsum(x):
    n = x.shape[1]
    lane = lax.broadcasted_iota(jnp.int32, x.shape, 1)
    shift = 1
    while shift < n:
        rolled = pltpu.roll(x, shift, 1)
        x = x + jnp.where(lane >= shift, rolled, 0.0)
        shift *= 2
    return x


def _topk_mask(v1, v2, v1l, v2l, noise_l):
    lr = noise_l.shape[0]
    nblocks = lr // BR
    return pl.pallas_call(
        _main_body,
        grid=(nblocks,),
        in_specs=[
            pl.BlockSpec((NSUB, DIM), lambda i: (0, 0)),
            pl.BlockSpec((NSUB, DIM), lambda i: (0, 0)),
            pl.BlockSpec((BR, DIM), lambda i: (i, 0)),
            pl.BlockSpec((BR, DIM), lambda i: (i, 0)),
            pl.BlockSpec((BR, NSUB), lambda i: (i, 0)),
        ],
        out_specs=pl.BlockSpec((BR, NSUB), lambda i: (i, 0)),
        out_shape=jax.ShapeDtypeStruct((lr, NSUB), jnp.float32),
        compiler_params=pltpu.CompilerParams(
            dimension_semantics=("arbitrary",),
            vmem_limit_bytes=64 * 1024 * 1024,
        ),
    )(v1, v2, v1l, v2l, noise_l)


@functools.cache
def _get_mesh():
    devs = jax.devices()
    nd = 2 if len(devs) >= 2 and NSUB % (2 * BR) == 0 else 1
    return jax.make_mesh((nd,), ("x",), devices=devs[:nd]), nd


def kernel(idx, emb1, emb2, W1, b1, W2, b2, noise):
    mesh, nd = _get_mesh()
    Ps = jax.sharding.PartitionSpec
    lr = NSUB // nd

    def local_fn(idx, emb1, emb2, W1, b1, W2, b2, noise_l):
        g1, g2 = _make_sc_gather()(emb1, emb2, idx)
        v1, v2 = _linear(g1, g2, W1, b1, W2, b2)
        ro = lax.axis_index("x") * lr
        v1l = lax.dynamic_slice(v1, (ro, 0), (lr, DIM))
        v2l = lax.dynamic_slice(v2, (ro, 0), (lr, DIM))
        return _topk_mask(v1, v2, v1l, v2l, noise_l)

    if nd == 1:
        return local_fn(idx, emb1, emb2, W1, b1, W2, b2, noise)

    rep = jax.sharding.NamedSharding(mesh, Ps())
    rows = jax.sharding.NamedSharding(mesh, Ps("x", None))
    idx, emb1, emb2, W1, b1, W2, b2 = (
        jax.device_put(v, rep) for v in (idx, emb1, emb2, W1, b1, W2, b2))
    noise = jax.device_put(noise, rows)
    return jax.shard_map(
        local_fn, mesh=mesh,
        in_specs=(Ps(), Ps(), Ps(), Ps(), Ps(), Ps(), Ps(), Ps("x", None)),
        out_specs=Ps("x", None), check_vma=False,
    )(idx, emb1, emb2, W1, b1, W2, b2, noise)


# speedup vs baseline: 2.8621x; 2.8621x over previous
import functools

import jax
import jax.numpy as jnp
import numpy as np
from jax import lax
from jax.experimental import pallas as pl
from jax.experimental.pallas import tpu as pltpu
from jax.experimental.pallas import tpu_sc as plsc

NSUB = 4096
DIM = 256
K = 16
ALPHA = 3.0
BR = 256
NCH = NSUB // 128
_DOT_DIMS = (((1,), (1,)), ((), ()))
_DOT_NN = (((1,), (0,)), ((), ()))


@functools.cache
def _make_sc_gather():
    info = plsc.get_sparse_core_info()
    nc, ns = info.num_cores, info.num_subcores
    nw = nc * ns
    bpw = NSUB // nw

    @functools.partial(
        pl.kernel,
        mesh=plsc.VectorSubcoreMesh(core_axis_name="c", subcore_axis_name="s"),
        out_type=[
            jax.ShapeDtypeStruct((NSUB, DIM), jnp.float32),
            jax.ShapeDtypeStruct((NSUB, DIM), jnp.float32),
        ],
        scratch_types=[
            pltpu.VMEM((bpw,), jnp.int32),
            pltpu.VMEM((bpw, DIM), jnp.float32),
            pltpu.VMEM((bpw, DIM), jnp.float32),
            pltpu.SemaphoreType.DMA,
            pltpu.SemaphoreType.DMA,
        ],
    )
    def sc_gather(emb1_hbm, emb2_hbm, idx_hbm, g1_hbm, g2_hbm,
                  idx_v, rows1_v, rows2_v, sem1, sem2):
        wid = lax.axis_index("s") * nc + lax.axis_index("c")
        base = wid * bpw
        pltpu.sync_copy(idx_hbm.at[pl.ds(base, bpw)], idx_v)
        c1 = pltpu.async_copy(emb1_hbm.at[idx_v], rows1_v, sem1)
        c2 = pltpu.async_copy(emb2_hbm.at[idx_v], rows2_v, sem2)
        c1.wait()
        c2.wait()
        pltpu.sync_copy(rows1_v, g1_hbm.at[pl.ds(base, bpw)])
        pltpu.sync_copy(rows2_v, g2_hbm.at[pl.ds(base, bpw)])

    return sc_gather


def _linear_body(g1_ref, g2_ref, w1_ref, b1_ref, w2_ref, b2_ref,
                 v1_ref, v2_ref):
    v1_ref[...] = jnp.tanh(ALPHA * (
        lax.dot_general(g1_ref[...], w1_ref[...], _DOT_DIMS,
                        preferred_element_type=jnp.float32)
        + b1_ref[...]))
    v2_ref[...] = jnp.tanh(ALPHA * (
        lax.dot_general(g2_ref[...], w2_ref[...], _DOT_DIMS,
                        preferred_element_type=jnp.float32)
        + b2_ref[...]))


def _linear(g1, g2, W1, b1, W2, b2):
    return pl.pallas_call(
        _linear_body,
        out_shape=[
            jax.ShapeDtypeStruct((NSUB, DIM), jnp.float32),
            jax.ShapeDtypeStruct((NSUB, DIM), jnp.float32),
        ],
    )(g1, g2, W1, b1.reshape(1, DIM), W2, b2.reshape(1, DIM))


def _count_gt(y, thr):
    return jnp.sum((y > thr).astype(jnp.float32), axis=1, keepdims=True)


def _main_body(v1_ref, v2_ref, v1l_ref, v2l_ref, noise_ref, t128_ref,
               s32_ref, bml_ref, srep_ref, out_ref):
    a = (lax.dot_general(v1l_ref[...], v2_ref[...], _DOT_DIMS,
                         preferred_element_type=jnp.float32)
         - lax.dot_general(v2l_ref[...], v1_ref[...], _DOT_DIMS,
                           preferred_element_type=jnp.float32))
    adj = jnp.maximum(jnp.tanh(ALPHA * a), 0.0)
    y = adj + noise_ref[...]

    x = y
    ms = []
    for _ in range(K):
        m = jnp.max(x, axis=1, keepdims=True)
        ms.append(m)
        x = jnp.where(x == m, -1.0, x)
    msmat = jnp.concatenate(ms, axis=1)

    lanes16 = lax.broadcasted_iota(jnp.int32, (BR, K), 1)
    lo = jnp.zeros((BR, 1), jnp.int32)
    hi = jnp.full((BR, 1), K, jnp.int32)
    for _ in range(4):
        mid = (lo + hi) // 2
        cand = jnp.sum(jnp.where(lanes16 == mid, msmat, 0.0),
                       axis=1, keepdims=True)
        ok = _count_gt(y, cand) < K
        lo = jnp.where(ok, mid, lo)
        hi = jnp.where(ok, hi, mid)
    t = jnp.sum(jnp.where(lanes16 == lo, msmat, 0.0), axis=1, keepdims=True)

    gt = y > t
    quota = K - jnp.sum(gt.astype(jnp.float32), axis=1, keepdims=True)
    eq = y == t
    ef = eq.astype(jnp.bfloat16)
    t128 = t128_ref[...]
    parts = [
        lax.dot_general(ef[:, c * 128:(c + 1) * 128], t128, _DOT_NN,
                        preferred_element_type=jnp.float32)
        for c in range(NCH)
    ]
    pfx = jnp.concatenate(parts, axis=1)
    csum = lax.dot_general(ef, s32_ref[...], _DOT_NN,
                           preferred_element_type=jnp.float32)
    offs = lax.dot_general(csum, bml_ref[...], _DOT_NN,
                           preferred_element_type=jnp.float32)
    offsb = lax.dot_general(offs, srep_ref[...], _DOT_NN,
                            preferred_element_type=jnp.float32)
    sel = gt | (eq & (pfx + offsb <= quota))
    out_ref[...] = jnp.where(sel, adj, 0.0)


@functools.cache
def _scan_mats():
    l = np.arange(128)
    t128 = (l[:, None] <= l[None, :]).astype(np.float32)
    i = np.arange(NSUB)
    c = np.arange(NCH)
    s32 = ((i[:, None] // 128) == c[None, :]).astype(np.float32)
    bml = (c[:, None] < c[None, :]).astype(np.float32)
    srep = (c[:, None] == (i[None, :] // 128)).astype(np.float32)
    return (jnp.asarray(t128, dtype=jnp.bfloat16),
            jnp.asarray(s32, dtype=jnp.bfloat16),
            jnp.asarray(bml), jnp.asarray(srep))


def _topk_mask(v1, v2, v1l, v2l, noise_l):
    lr = noise_l.shape[0]
    nblocks = lr // BR
    return pl.pallas_call(
        _main_body,
        grid=(nblocks,),
        in_specs=[
            pl.BlockSpec((NSUB, DIM), lambda i: (0, 0)),
            pl.BlockSpec((NSUB, DIM), lambda i: (0, 0)),
            pl.BlockSpec((BR, DIM), lambda i: (i, 0)),
            pl.BlockSpec((BR, DIM), lambda i: (i, 0)),
            pl.BlockSpec((BR, NSUB), lambda i: (i, 0)),
            pl.BlockSpec((128, 128), lambda i: (0, 0)),
            pl.BlockSpec((NSUB, NCH), lambda i: (0, 0)),
            pl.BlockSpec((NCH, NCH), lambda i: (0, 0)),
            pl.BlockSpec((NCH, NSUB), lambda i: (0, 0)),
        ],
        out_specs=pl.BlockSpec((BR, NSUB), lambda i: (i, 0)),
        out_shape=jax.ShapeDtypeStruct((lr, NSUB), jnp.float32),
        compiler_params=pltpu.CompilerParams(
            dimension_semantics=("arbitrary",),
            vmem_limit_bytes=64 * 1024 * 1024,
        ),
    )(v1, v2, v1l, v2l, noise_l, *_scan_mats())


def kernel(idx, emb1, emb2, W1, b1, W2, b2, noise):
    g1, g2 = _make_sc_gather()(emb1, emb2, idx)
    v1, v2 = _linear(g1, g2, W1, b1, W2, b2)
    return _topk_mask(v1, v2, v1, v2, noise)
